# Initial kernel scaffold; baseline (speedup 1.0000x reference)
import jax
import jax.numpy as jnp
from jax.experimental import pallas as pl


def kernel(x, edge_index, W1, a_src1, a_dst1, b1, W2, a_src2, a_dst2, b2):
    raise NotImplementedError("write your pallas kernel here")


# speedup vs baseline: 1.1503x; 1.1503x over previous
import jax
import jax.numpy as jnp
from jax.experimental import pallas as pl

N = 10000
BM = 1000


def _mm_body(x_ref, w_ref, o_ref):
    o_ref[...] = jnp.dot(x_ref[...], w_ref[...], preferred_element_type=jnp.float32)


def _matmul(x, w):
    M, K = x.shape
    _, Nc = w.shape
    return pl.pallas_call(
        _mm_body,
        grid=(M // BM,),
        in_specs=[
            pl.BlockSpec((BM, K), lambda i: (i, 0)),
            pl.BlockSpec((K, Nc), lambda i: (0, 0)),
        ],
        out_specs=pl.BlockSpec((BM, Nc), lambda i: (i, 0)),
        out_shape=jax.ShapeDtypeStruct((M, Nc), jnp.float32),
    )(x, w)


def _gat_layer(x, src, dst, W, a_src, a_dst, b):
    z = _matmul(x, W)
    alpha_s = z @ a_src
    alpha_d = z @ a_dst
    e = jax.nn.leaky_relu(alpha_s[src] + alpha_d[dst], negative_slope=0.2)
    m = jax.ops.segment_max(e, dst, num_segments=N)
    m = jnp.where(jnp.isfinite(m), m, 0.0)
    ex = jnp.exp(e - m[dst])
    s = jax.ops.segment_sum(ex, dst, num_segments=N)
    coef = ex / (s[dst] + 1e-16)
    out = jax.ops.segment_sum(coef[:, None] * z[src], dst, num_segments=N)
    return out + b


def kernel(x, edge_index, W1, a_src1, a_dst1, b1, W2, a_src2, a_dst2, b2):
    src = edge_index[0]
    dst = edge_index[1]
    h = _gat_layer(x, src, dst, W1, a_src1, a_dst1, b1)
    h = jax.nn.relu(h)
    h = _gat_layer(h, src, dst, W2, a_src2, a_dst2, b2)
    h = jax.nn.relu(h)
    return h


# speedup vs baseline: 6.7158x; 5.8384x over previous
import functools

import jax
import jax.numpy as jnp
from jax import lax
from jax.experimental import pallas as pl
from jax.experimental.pallas import tpu as pltpu
from jax.experimental.pallas import tpu_sc as plsc

N = 10000
E = 320000
NW = 32
NL = 320
NPAD = NW * NL

CB = 2560
FLUSH = 2048
CAP = E + CB

CA = 2048
ROWS = 32

BM = 1000


def _mm_body(x_ref, w_ref, o_ref):
    o_ref[...] = jnp.dot(x_ref[...], w_ref[...], preferred_element_type=jnp.float32)


def _matmul(x, w):
    M, K = x.shape
    _, Nc = w.shape
    return pl.pallas_call(
        _mm_body,
        grid=(M // BM,),
        in_specs=[
            pl.BlockSpec((BM, K), lambda i: (i, 0)),
            pl.BlockSpec((K, Nc), lambda i: (0, 0)),
        ],
        out_specs=pl.BlockSpec((BM, Nc), lambda i: (i, 0)),
        out_shape=jax.ShapeDtypeStruct((M, Nc), jnp.float32),
    )(x, w)


def _sc_mesh():
    return plsc.VectorSubcoreMesh(core_axis_name="c", subcore_axis_name="s")


_SC_PARAMS = pltpu.CompilerParams(needs_layout_passes=False)


def _wid():
    return lax.axis_index("s") * 2 + lax.axis_index("c")


def _bucketize(edge_index):
    @functools.partial(
        pl.kernel,
        out_type=(
            jax.ShapeDtypeStruct((NW * CAP,), jnp.int32),
            jax.ShapeDtypeStruct((NW * 16,), jnp.int32),
        ),
        mesh=_sc_mesh(),
        compiler_params=_SC_PARAMS,
        scratch_types=[
            pltpu.VMEM((CB,), jnp.int32),
            pltpu.VMEM((CB,), jnp.int32),
            pltpu.VMEM((FLUSH + 32,), jnp.int32),
            pltpu.VMEM((16,), jnp.int32),
        ],
    )
    def bucketize(ei_hbm, bedges_hbm, counts_hbm, srcc, dstc, stage, cstage):
        wid = _wid()
        base = wid * NL

        def chunk_body(c, carry):
            fill, wpos = carry
            cb = c * CB
            pltpu.sync_copy(ei_hbm.at[0, pl.ds(cb, CB)], srcc)
            pltpu.sync_copy(ei_hbm.at[1, pl.ds(cb, CB)], dstc)

            def vreg_body(j, carry2):
                fill, wpos = carry2
                sv = srcc[pl.ds(j * 16, 16)]
                dv = dstc[pl.ds(j * 16, 16)]
                mine = (dv >= base) & (dv < base + NL)
                packed = sv * 512 + (dv - base)
                minei = jnp.where(mine, 1, 0)
                pos = fill + plsc.cumsum(minei) - 1
                idx = jnp.where(mine, pos, FLUSH + 16)
                plsc.store_scatter(stage, [idx], packed)
                fill = fill + jnp.sum(minei)

                def do_flush(args):
                    fill, wpos = args
                    off = pl.multiple_of(wid * CAP + wpos, 8)
                    pltpu.sync_copy(stage.at[pl.ds(0, FLUSH)],
                                    bedges_hbm.at[pl.ds(off, FLUSH)])
                    resid = stage[pl.ds(FLUSH, 16)]
                    stage[pl.ds(0, 16)] = resid
                    return fill - FLUSH, wpos + FLUSH

                return lax.cond(fill >= FLUSH, do_flush, lambda a: a, (fill, wpos))

            return lax.fori_loop(0, CB // 16, vreg_body, (fill, wpos))

        fill, wpos = lax.fori_loop(0, E // CB, chunk_body,
                                   (jnp.int32(0), jnp.int32(0)))
        off = pl.multiple_of(wid * CAP + wpos, 8)
        pltpu.sync_copy(stage.at[pl.ds(0, FLUSH)],
                        bedges_hbm.at[pl.ds(off, FLUSH)])
        cstage[...] = jnp.full((16,), wpos + fill, jnp.int32)
        pltpu.sync_copy(cstage, counts_hbm.at[pl.ds(pl.multiple_of(wid * 16, 8), 16)])

    return bucketize(edge_index)


def _make_aggregate(D):
    @functools.partial(
        pl.kernel,
        out_type=jax.ShapeDtypeStruct((NPAD, D), jnp.float32),
        mesh=_sc_mesh(),
        compiler_params=_SC_PARAMS,
        scratch_types=[
            pltpu.VMEM((N,), jnp.float32),
            pltpu.VMEM((NL,), jnp.float32),
            pltpu.VMEM((NL + 16,), jnp.float32),
            pltpu.VMEM((NL + 16,), jnp.float32),
            pltpu.VMEM((NL * 16,), jnp.float32),
            pltpu.VMEM((NL, D), jnp.float32),
            pltpu.VMEM((CA + 16,), jnp.int32),
            pltpu.VMEM((CA + 16,), jnp.int32),
            pltpu.VMEM((CA,), jnp.int32),
            pltpu.VMEM((CA + 16,), jnp.float32),
            pltpu.VMEM((ROWS, D), jnp.float32),
            pltpu.VMEM((D,), jnp.float32),
            pltpu.VMEM((16,), jnp.int32),
            pltpu.SemaphoreType.DMA,
        ],
    )
    def aggregate(z_hbm, as_hbm, ad_hbm, bedges_hbm, counts_hbm, b_hbm, out_hbm,
                  asl, adl, m, s, priv, acc, pbuf, dlbuf, srcbuf, vbuf, rowbuf,
                  biasl, cbuf, sem):
        wid = _wid()
        base = wid * NL
        iota16 = lax.iota(jnp.int32, 16)
        lane0 = iota16 == 0

        pltpu.sync_copy(as_hbm, asl)
        pltpu.sync_copy(ad_hbm.at[pl.ds(pl.multiple_of(base, 8), NL)], adl)
        pltpu.sync_copy(b_hbm, biasl)
        pltpu.sync_copy(counts_hbm.at[pl.ds(pl.multiple_of(wid * 16, 8), 16)], cbuf)
        cnt = cbuf[...][0]
        nch = (cnt + (CA - 1)) // CA

        def fill_vec(ref, n16, value):
            def body(j, _):
                ref[pl.ds(j * 16, 16)] = jnp.full((16,), value, jnp.float32)
                return 0
            lax.fori_loop(0, n16, body, 0)

        fill_vec(priv, NL, -1e30)

        def init_acc(i, _):
            for k in range(D // 16):
                acc[i, pl.ds(k * 16, 16)] = jnp.zeros((16,), jnp.float32)
            return 0
        lax.fori_loop(0, NL, init_acc, 0)

        def load_chunk(cb):
            pltpu.sync_copy(bedges_hbm.at[pl.ds(pl.multiple_of(wid * CAP + cb, 8), CA)],
                            pbuf.at[pl.ds(0, CA)])

        def unpack_vreg(j, cb):
            pk = pbuf[pl.ds(j * 16, 16)]
            valid = (cb + j * 16 + iota16) < cnt
            srcv = jnp.where(valid, lax.shift_right_logical(pk, 9), 0)
            dstlv = jnp.where(valid, pk & 511, 0)
            return valid, srcv, dstlv

        def edge_e(srcv, dstlv, valid):
            asg = plsc.load_gather(asl, [srcv], mask=valid)
            adg = plsc.load_gather(adl, [dstlv], mask=valid)
            t = asg + adg
            return jnp.maximum(t, t * 0.2)

        def p1_chunk(c, _):
            cb = c * CA
            load_chunk(cb)

            def vbody(j, _):
                valid, srcv, dstlv = unpack_vreg(j, cb)
                e = jnp.where(valid, edge_e(srcv, dstlv, valid), -1e30)
                pidx = dstlv * 16 + iota16
                cur = plsc.load_gather(priv, [pidx])
                plsc.store_scatter(priv, [pidx], jnp.maximum(cur, e))
                return 0
            lax.fori_loop(0, CA // 16, vbody, 0)
            return 0
        lax.fori_loop(0, nch, p1_chunk, 0)

        def red_max(i, _):
            v = priv[pl.ds(i * 16, 16)]
            mx = jnp.max(v)
            idx = jnp.where(lane0, i, NL)
            plsc.store_scatter(m, [idx], jnp.full((16,), mx, jnp.float32))
            return 0
        lax.fori_loop(0, NL, red_max, 0)

        fill_vec(priv, NL, 0.0)

        def p2_chunk(c, _):
            cb = c * CA
            load_chunk(cb)

            def vbody(j, _):
                valid, srcv, dstlv = unpack_vreg(j, cb)
                e = edge_e(srcv, dstlv, valid)
                mg = plsc.load_gather(m, [dstlv], mask=valid)
                ex = jnp.where(valid, jnp.exp(e - mg), 0.0)
                pidx = dstlv * 16 + iota16
                cur = plsc.load_gather(priv, [pidx])
                plsc.store_scatter(priv, [pidx], cur + ex)
                return 0
            lax.fori_loop(0, CA // 16, vbody, 0)
            return 0
        lax.fori_loop(0, nch, p2_chunk, 0)

        def red_sum(i, _):
            v = priv[pl.ds(i * 16, 16)]
            sm = jnp.sum(v)
            idx = jnp.where(lane0, i, NL)
            plsc.store_scatter(s, [idx], jnp.full((16,), sm, jnp.float32))
            return 0
        lax.fori_loop(0, NL, red_sum, 0)

        def p3_chunk(c, _):
            cb = c * CA
            load_chunk(cb)

            def vbody(j, _):
                valid, srcv, dstlv = unpack_vreg(j, cb)
                e = edge_e(srcv, dstlv, valid)
                mg = plsc.load_gather(m, [dstlv], mask=valid)
                sg = plsc.load_gather(s, [dstlv], mask=valid)
                ex = jnp.where(valid, jnp.exp(e - mg), 0.0)
                coef = ex / (sg + 1e-16)
                coef = jnp.where(valid, coef, 0.0)
                dlbuf[pl.ds(j * 16, 16)] = dstlv
                srcbuf[pl.ds(j * 16, 16)] = srcv
                vbuf[pl.ds(j * 16, 16)] = coef
                return 0
            lax.fori_loop(0, CA // 16, vbody, 0)

            ne = jnp.minimum(CA, cnt - cb)
            nsub = (ne + (ROWS - 1)) // ROWS

            def sub_body(sb, _):
                rb = sb * ROWS
                pltpu.async_copy(z_hbm.at[srcbuf.at[pl.ds(rb, ROWS)]],
                                 rowbuf, sem).wait()
                nr = jnp.minimum(ROWS, ne - rb)

                def row_body(r, _):
                    cf = vbuf[pl.ds(rb + r, 16)][0]
                    dl = dlbuf[pl.ds(rb + r, 16)][0]
                    for k in range(D // 16):
                        sl = pl.ds(k * 16, 16)
                        acc[dl, sl] = acc[dl, sl] + rowbuf[r, sl] * cf
                    return 0
                lax.fori_loop(0, nr, row_body, 0)
                return 0
            lax.fori_loop(0, nsub, sub_body, 0)
            return 0
        lax.fori_loop(0, nch, p3_chunk, 0)

        def out_body(i, _):
            for k in range(D // 16):
                sl = pl.ds(k * 16, 16)
                acc[i, sl] = jnp.maximum(acc[i, sl] + biasl[sl], 0.0)
            return 0
        lax.fori_loop(0, NL, out_body, 0)

        pltpu.sync_copy(acc, out_hbm.at[pl.ds(base, NL)])

    return aggregate


_aggregate_256 = _make_aggregate(256)
_aggregate_128 = _make_aggregate(128)


def kernel(x, edge_index, W1, a_src1, a_dst1, b1, W2, a_src2, a_dst2, b2):
    ei = edge_index.astype(jnp.int32)
    bedges, counts = _bucketize(ei)

    W1e = jnp.concatenate(
        [W1, (W1 @ a_src1)[:, None], (W1 @ a_dst1)[:, None]], axis=1)
    ze1 = _matmul(x, W1e)
    z1 = ze1[:, :256]
    as1 = ze1[:, 256]
    ad1 = jnp.pad(ze1[:, 257], (0, NPAD - N))
    h1 = _aggregate_256(z1, as1, ad1, bedges, counts, b1)[:N]

    W2e = jnp.concatenate(
        [W2, (W2 @ a_src2)[:, None], (W2 @ a_dst2)[:, None]], axis=1)
    ze2 = _matmul(h1, W2e)
    z2 = ze2[:, :128]
    as2 = ze2[:, 128]
    ad2 = jnp.pad(ze2[:, 129], (0, NPAD - N))
    h2 = _aggregate_128(z2, as2, ad2, bedges, counts, b2)[:N]
    return h2


# speedup vs baseline: 9.5451x; 1.4213x over previous
import functools

import jax
import jax.numpy as jnp
from jax import lax
from jax.experimental import pallas as pl
from jax.experimental.pallas import tpu as pltpu
from jax.experimental.pallas import tpu_sc as plsc

N = 10000
E = 320000
NW = 32
NL = 320
NPAD = NW * NL

CB = 2560
FLUSH = 2048
CAP = E + CB

CA = 2048
ROWS = 32

BM = 1000


def _mm_body(x_ref, w_ref, o_ref):
    o_ref[...] = jnp.dot(x_ref[...], w_ref[...], preferred_element_type=jnp.float32)


def _matmul(x, w):
    M, K = x.shape
    _, Nc = w.shape
    return pl.pallas_call(
        _mm_body,
        grid=(M // BM,),
        in_specs=[
            pl.BlockSpec((BM, K), lambda i: (i, 0)),
            pl.BlockSpec((K, Nc), lambda i: (0, 0)),
        ],
        out_specs=pl.BlockSpec((BM, Nc), lambda i: (i, 0)),
        out_shape=jax.ShapeDtypeStruct((M, Nc), jnp.float32),
    )(x, w)


def _sc_mesh():
    return plsc.VectorSubcoreMesh(core_axis_name="c", subcore_axis_name="s")


_SC_PARAMS = pltpu.CompilerParams(needs_layout_passes=False)


def _wid():
    return lax.axis_index("s") * 2 + lax.axis_index("c")


def _bucketize(edge_index):
    @functools.partial(
        pl.kernel,
        out_type=(
            jax.ShapeDtypeStruct((NW * CAP,), jnp.int32),
            jax.ShapeDtypeStruct((NW * 16,), jnp.int32),
        ),
        mesh=_sc_mesh(),
        compiler_params=_SC_PARAMS,
        scratch_types=[
            pltpu.VMEM((CB,), jnp.int32),
            pltpu.VMEM((CB,), jnp.int32),
            pltpu.VMEM((FLUSH + 32,), jnp.int32),
            pltpu.VMEM((16,), jnp.int32),
        ],
    )
    def bucketize(ei_hbm, bedges_hbm, counts_hbm, srcc, dstc, stage, cstage):
        wid = _wid()
        base = wid * NL

        def chunk_body(c, carry):
            fill, wpos = carry
            cb = c * CB
            pltpu.sync_copy(ei_hbm.at[0, pl.ds(cb, CB)], srcc)
            pltpu.sync_copy(ei_hbm.at[1, pl.ds(cb, CB)], dstc)

            def vreg_body(j, carry2):
                fill, wpos = carry2
                sv = srcc[pl.ds(j * 16, 16)]
                dv = dstc[pl.ds(j * 16, 16)]
                mine = (dv >= base) & (dv < base + NL)
                packed = sv * 512 + (dv - base)
                minei = jnp.where(mine, 1, 0)
                pos = fill + plsc.cumsum(minei) - 1
                idx = jnp.where(mine, pos, FLUSH + 16)
                plsc.store_scatter(stage, [idx], packed)
                fill = fill + jnp.sum(minei)

                def do_flush(args):
                    fill, wpos = args
                    off = pl.multiple_of(wid * CAP + wpos, 8)
                    pltpu.sync_copy(stage.at[pl.ds(0, FLUSH)],
                                    bedges_hbm.at[pl.ds(off, FLUSH)])
                    resid = stage[pl.ds(FLUSH, 16)]
                    stage[pl.ds(0, 16)] = resid
                    return fill - FLUSH, wpos + FLUSH

                return lax.cond(fill >= FLUSH, do_flush, lambda a: a, (fill, wpos))

            return lax.fori_loop(0, CB // 16, vreg_body, (fill, wpos))

        fill, wpos = lax.fori_loop(0, E // CB, chunk_body,
                                   (jnp.int32(0), jnp.int32(0)))
        off = pl.multiple_of(wid * CAP + wpos, 8)
        pltpu.sync_copy(stage.at[pl.ds(0, FLUSH)],
                        bedges_hbm.at[pl.ds(off, FLUSH)])
        cstage[...] = jnp.full((16,), wpos + fill, jnp.int32)
        pltpu.sync_copy(cstage, counts_hbm.at[pl.ds(pl.multiple_of(wid * 16, 8), 16)])

    return bucketize(edge_index)


def _make_aggregate(D):
    @functools.partial(
        pl.kernel,
        out_type=jax.ShapeDtypeStruct((NPAD, D), jnp.float32),
        mesh=_sc_mesh(),
        compiler_params=_SC_PARAMS,
        scratch_types=[
            pltpu.VMEM((N,), jnp.float32),
            pltpu.VMEM((NL,), jnp.float32),
            pltpu.VMEM((NL + 16,), jnp.float32),
            pltpu.VMEM((16 * NL,), jnp.float32),
            pltpu.VMEM((NL, D), jnp.float32),
            pltpu.VMEM((CA + 16,), jnp.int32),
            pltpu.VMEM((CA + 16,), jnp.int32),
            pltpu.VMEM((CA,), jnp.int32),
            pltpu.VMEM((CA + 16,), jnp.float32),
            pltpu.VMEM((2, ROWS, D), jnp.float32),
            pltpu.VMEM((D,), jnp.float32),
            pltpu.VMEM((16,), jnp.int32),
            pltpu.SemaphoreType.DMA,
            pltpu.SemaphoreType.DMA,
        ],
    )
    def aggregate(z_hbm, as_hbm, ad_hbm, bedges_hbm, counts_hbm, b_hbm, out_hbm,
                  asl, adl, s, priv, acc, pbuf, dlbuf, srcbuf, vbuf, rowbuf,
                  biasl, cbuf, sem0, sem1):
        wid = _wid()
        base = wid * NL
        iota16 = lax.iota(jnp.int32, 16)

        pltpu.sync_copy(as_hbm, asl)
        pltpu.sync_copy(ad_hbm.at[pl.ds(pl.multiple_of(base, 8), NL)], adl)
        pltpu.sync_copy(b_hbm, biasl)
        pltpu.sync_copy(counts_hbm.at[pl.ds(pl.multiple_of(wid * 16, 8), 16)], cbuf)
        cnt = cbuf[...][0]
        nch = (cnt + (CA - 1)) // CA

        zero16 = jnp.zeros((16,), jnp.float32)

        def zpriv(j, _):
            priv[pl.ds(j * 16, 16)] = zero16
            return 0
        lax.fori_loop(0, NL, zpriv, 0)

        def init_acc(i, _):
            for k in range(D // 16):
                acc[i, pl.ds(k * 16, 16)] = zero16
            return 0
        lax.fori_loop(0, NL, init_acc, 0)

        def chunk_body(c, _):
            cb = c * CA
            pltpu.sync_copy(
                bedges_hbm.at[pl.ds(pl.multiple_of(wid * CAP + cb, 8), CA)],
                pbuf.at[pl.ds(0, CA)])

            def vbody(j, _):
                pk = pbuf[pl.ds(j * 16, 16)]
                valid = (cb + j * 16 + iota16) < cnt
                srcv = jnp.where(valid, lax.shift_right_logical(pk, 9), 0)
                dstlv = jnp.where(valid, pk & 511, 0)
                asg = plsc.load_gather(asl, [srcv], mask=valid)
                adg = plsc.load_gather(adl, [dstlv], mask=valid)
                t = asg + adg
                e = jnp.maximum(t, t * 0.2)
                ex = jnp.where(valid, jnp.exp(e), 0.0)
                pidx = iota16 * NL + dstlv
                cur = plsc.load_gather(priv, [pidx])
                plsc.store_scatter(priv, [pidx], cur + ex)
                dlbuf[pl.ds(j * 16, 16)] = dstlv
                srcbuf[pl.ds(j * 16, 16)] = srcv
                vbuf[pl.ds(j * 16, 16)] = ex
                return 0
            lax.fori_loop(0, CA // 16, vbody, 0)

            ne = jnp.minimum(CA, cnt - cb)
            nsub = (ne + (ROWS - 1)) // ROWS

            def issue(sb):
                idx = srcbuf.at[pl.ds(sb * ROWS, ROWS)]

                def even(_):
                    pltpu.make_async_copy(z_hbm.at[idx], rowbuf.at[0], sem0
                                          ).start()
                    return 0

                def odd(_):
                    pltpu.make_async_copy(z_hbm.at[idx], rowbuf.at[1], sem1
                                          ).start()
                    return 0
                lax.cond(sb % 2 == 0, even, odd, 0)

            def drain(sb):
                idx = srcbuf.at[pl.ds(0, ROWS)]

                def even(_):
                    pltpu.make_async_copy(z_hbm.at[idx], rowbuf.at[0], sem0
                                          ).wait()
                    return 0

                def odd(_):
                    pltpu.make_async_copy(z_hbm.at[idx], rowbuf.at[1], sem1
                                          ).wait()
                    return 0
                lax.cond(sb % 2 == 0, even, odd, 0)

            issue(0)

            def sub_body(sb, _):
                def more(_):
                    issue(sb + 1)
                    return 0
                lax.cond(sb + 1 < nsub, more, lambda x: x, 0)
                drain(sb)
                par = sb % 2
                rb = sb * ROWS

                def row_body(r, _):
                    cf = vbuf[pl.ds(rb + r, 16)][0]
                    dl = dlbuf[pl.ds(rb + r, 16)][0]
                    for k in range(D // 16):
                        sl = pl.ds(k * 16, 16)
                        plsc.addupdate(acc.at[dl, sl], rowbuf[par, r, sl] * cf)
                    return 0
                lax.fori_loop(0, jnp.minimum(ROWS, ne - rb), row_body, 0)
                return 0
            lax.fori_loop(0, nsub, sub_body, 0)
            return 0
        lax.fori_loop(0, nch, chunk_body, 0)

        def red_sum(c, _):
            v = priv[pl.ds(c * 16, 16)]
            for l in range(1, 16):
                v = v + priv[pl.ds(l * NL + c * 16, 16)]
            s[pl.ds(c * 16, 16)] = 1.0 / (v + 1e-16)
            return 0
        lax.fori_loop(0, NL // 16, red_sum, 0)

        def out_body(i, _):
            si = s[pl.ds(i, 16)][0]
            for k in range(D // 16):
                sl = pl.ds(k * 16, 16)
                acc[i, sl] = jnp.maximum(acc[i, sl] * si + biasl[sl], 0.0)
            return 0
        lax.fori_loop(0, NL, out_body, 0)

        pltpu.sync_copy(acc, out_hbm.at[pl.ds(base, NL)])

    return aggregate


_aggregate_256 = _make_aggregate(256)
_aggregate_128 = _make_aggregate(128)


def kernel(x, edge_index, W1, a_src1, a_dst1, b1, W2, a_src2, a_dst2, b2):
    ei = edge_index.astype(jnp.int32)
    bedges, counts = _bucketize(ei)

    W1e = jnp.concatenate(
        [W1, (W1 @ a_src1)[:, None], (W1 @ a_dst1)[:, None]], axis=1)
    ze1 = _matmul(x, W1e)
    z1 = ze1[:, :256]
    as1 = ze1[:, 256]
    ad1 = jnp.pad(ze1[:, 257], (0, NPAD - N))
    h1 = _aggregate_256(z1, as1, ad1, bedges, counts, b1)[:N]

    W2e = jnp.concatenate(
        [W2, (W2 @ a_src2)[:, None], (W2 @ a_dst2)[:, None]], axis=1)
    ze2 = _matmul(h1, W2e)
    z2 = ze2[:, :128]
    as2 = ze2[:, 128]
    ad2 = jnp.pad(ze2[:, 129], (0, NPAD - N))
    h2 = _aggregate_128(z2, as2, ad2, bedges, counts, b2)[:N]
    return h2


# speedup vs baseline: 10.8687x; 1.1387x over previous
import functools

import jax
import jax.numpy as jnp
from jax import lax
from jax.experimental import pallas as pl
from jax.experimental.pallas import tpu as pltpu
from jax.experimental.pallas import tpu_sc as plsc

N = 10000
E = 320000
NW = 32
NL = 320
NPAD = NW * NL

CB = 2560
FLUSH = 2048
CAP = E + CB

CA = 2048
ROWS = 32

BM = 1000


def _mm_body(x_ref, w_ref, o_ref):
    o_ref[...] = jnp.dot(x_ref[...], w_ref[...], preferred_element_type=jnp.float32)


def _matmul(x, w):
    M, K = x.shape
    _, Nc = w.shape
    return pl.pallas_call(
        _mm_body,
        grid=(M // BM,),
        in_specs=[
            pl.BlockSpec((BM, K), lambda i: (i, 0)),
            pl.BlockSpec((K, Nc), lambda i: (0, 0)),
        ],
        out_specs=pl.BlockSpec((BM, Nc), lambda i: (i, 0)),
        out_shape=jax.ShapeDtypeStruct((M, Nc), jnp.float32),
    )(x, w)


def _sc_mesh():
    return plsc.VectorSubcoreMesh(core_axis_name="c", subcore_axis_name="s")


_SC_PARAMS = pltpu.CompilerParams(needs_layout_passes=False)


def _wid():
    return lax.axis_index("s") * 2 + lax.axis_index("c")


def _bucketize(edge_index):
    @functools.partial(
        pl.kernel,
        out_type=(
            jax.ShapeDtypeStruct((NW * CAP,), jnp.int32),
            jax.ShapeDtypeStruct((NW * 16,), jnp.int32),
        ),
        mesh=_sc_mesh(),
        compiler_params=_SC_PARAMS,
        scratch_types=[
            pltpu.VMEM((CB,), jnp.int32),
            pltpu.VMEM((CB,), jnp.int32),
            pltpu.VMEM((FLUSH + CB + 32,), jnp.int32),
            pltpu.VMEM((16,), jnp.int32),
        ],
    )
    def bucketize(ei_hbm, bedges_hbm, counts_hbm, srcc, dstc, stage, cstage):
        wid = _wid()
        base = wid * NL

        def chunk_body(c, carry):
            fill, wpos = carry
            cb = c * CB
            pltpu.sync_copy(ei_hbm.at[0, pl.ds(cb, CB)], srcc)
            pltpu.sync_copy(ei_hbm.at[1, pl.ds(cb, CB)], dstc)

            def vreg_body(j, fill):
                sv = srcc[pl.ds(j * 16, 16)]
                dv = dstc[pl.ds(j * 16, 16)]
                mine = (dv >= base) & (dv < base + NL)
                packed = sv * 512 + (dv - base)
                minei = jnp.where(mine, 1, 0)
                pos = fill + plsc.cumsum(minei) - 1
                idx = jnp.where(mine, pos, FLUSH + CB + 16)
                plsc.store_scatter(stage, [idx], packed)
                return fill + jnp.sum(minei)

            fill = lax.fori_loop(0, CB // 16, vreg_body, fill)

            def do_flush(args):
                fill, wpos = args
                off = pl.multiple_of(wid * CAP + wpos, 8)
                pltpu.sync_copy(stage.at[pl.ds(0, FLUSH)],
                                bedges_hbm.at[pl.ds(off, FLUSH)])
                resid = fill - FLUSH

                def shift(i, _):
                    stage[pl.ds(i * 16, 16)] = stage[pl.ds(FLUSH + i * 16, 16)]
                    return 0
                lax.fori_loop(0, (resid + 15) // 16, shift, 0)
                return resid, wpos + FLUSH

            return lax.while_loop(lambda a: a[0] >= FLUSH, do_flush, (fill, wpos))

        fill, wpos = lax.fori_loop(0, E // CB, chunk_body,
                                   (jnp.int32(0), jnp.int32(0)))
        off = pl.multiple_of(wid * CAP + wpos, 8)
        pltpu.sync_copy(stage.at[pl.ds(0, FLUSH)],
                        bedges_hbm.at[pl.ds(off, FLUSH)])
        cstage[...] = jnp.full((16,), wpos + fill, jnp.int32)
        pltpu.sync_copy(cstage, counts_hbm.at[pl.ds(pl.multiple_of(wid * 16, 8), 16)])

    return bucketize(edge_index)


def _make_aggregate(D):
    @functools.partial(
        pl.kernel,
        out_type=jax.ShapeDtypeStruct((NPAD, D), jnp.float32),
        mesh=_sc_mesh(),
        compiler_params=_SC_PARAMS,
        scratch_types=[
            pltpu.VMEM((N,), jnp.float32),
            pltpu.VMEM((NL,), jnp.float32),
            pltpu.VMEM((NL + 16,), jnp.float32),
            pltpu.VMEM((16 * NL,), jnp.float32),
            pltpu.VMEM((NL, D), jnp.float32),
            pltpu.VMEM((CA + 16,), jnp.int32),
            pltpu.VMEM((CA + 16,), jnp.int32),
            pltpu.VMEM((CA,), jnp.int32),
            pltpu.VMEM((CA + 16,), jnp.float32),
            pltpu.VMEM((2, ROWS, D), jnp.float32),
            pltpu.VMEM((D,), jnp.float32),
            pltpu.VMEM((16,), jnp.int32),
            pltpu.SemaphoreType.DMA,
            pltpu.SemaphoreType.DMA,
        ],
    )
    def aggregate(z_hbm, as_hbm, ad_hbm, bedges_hbm, counts_hbm, b_hbm, out_hbm,
                  asl, adl, s, priv, acc, pbuf, dlbuf, srcbuf, vbuf, rowbuf,
                  biasl, cbuf, sem0, sem1):
        wid = _wid()
        base = wid * NL
        iota16 = lax.iota(jnp.int32, 16)

        pltpu.sync_copy(as_hbm, asl)
        pltpu.sync_copy(ad_hbm.at[pl.ds(pl.multiple_of(base, 8), NL)], adl)
        pltpu.sync_copy(b_hbm, biasl)
        pltpu.sync_copy(counts_hbm.at[pl.ds(pl.multiple_of(wid * 16, 8), 16)], cbuf)
        cnt = cbuf[...][0]
        nch = (cnt + (CA - 1)) // CA

        zero16 = jnp.zeros((16,), jnp.float32)

        def zpriv(j, _):
            priv[pl.ds(j * 16, 16)] = zero16
            return 0
        lax.fori_loop(0, NL, zpriv, 0)

        def init_acc(i, _):
            for k in range(D // 16):
                acc[i, pl.ds(k * 16, 16)] = zero16
            return 0
        lax.fori_loop(0, NL, init_acc, 0)

        def chunk_body(c, _):
            cb = c * CA
            pltpu.sync_copy(
                bedges_hbm.at[pl.ds(pl.multiple_of(wid * CAP + cb, 8), CA)],
                pbuf.at[pl.ds(0, CA)])

            def vbody(j, _):
                pk = pbuf[pl.ds(j * 16, 16)]
                valid = (cb + j * 16 + iota16) < cnt
                srcv = jnp.where(valid, lax.shift_right_logical(pk, 9), 0)
                dstlv = jnp.where(valid, pk & 511, 0)
                asg = plsc.load_gather(asl, [srcv], mask=valid)
                adg = plsc.load_gather(adl, [dstlv], mask=valid)
                t = asg + adg
                e = jnp.maximum(t, t * 0.2)
                ex = jnp.where(valid, jnp.exp(e), 0.0)
                pidx = iota16 * NL + dstlv
                cur = plsc.load_gather(priv, [pidx])
                plsc.store_scatter(priv, [pidx], cur + ex)
                dlbuf[pl.ds(j * 16, 16)] = dstlv
                srcbuf[pl.ds(j * 16, 16)] = srcv
                vbuf[pl.ds(j * 16, 16)] = ex
                return 0
            lax.fori_loop(0, CA // 16, vbody, 0)

            ne = jnp.minimum(CA, cnt - cb)
            nsub = (ne + (ROWS - 1)) // ROWS

            def issue(sb):
                idx = srcbuf.at[pl.ds(sb * ROWS, ROWS)]

                def even(_):
                    pltpu.make_async_copy(z_hbm.at[idx], rowbuf.at[0], sem0
                                          ).start()
                    return 0

                def odd(_):
                    pltpu.make_async_copy(z_hbm.at[idx], rowbuf.at[1], sem1
                                          ).start()
                    return 0
                lax.cond(sb % 2 == 0, even, odd, 0)

            def drain(sb):
                idx = srcbuf.at[pl.ds(0, ROWS)]

                def even(_):
                    pltpu.make_async_copy(z_hbm.at[idx], rowbuf.at[0], sem0
                                          ).wait()
                    return 0

                def odd(_):
                    pltpu.make_async_copy(z_hbm.at[idx], rowbuf.at[1], sem1
                                          ).wait()
                    return 0
                lax.cond(sb % 2 == 0, even, odd, 0)

            issue(0)

            def sub_body(sb, _):
                def more(_):
                    issue(sb + 1)
                    return 0
                lax.cond(sb + 1 < nsub, more, lambda x: x, 0)
                drain(sb)
                par = sb % 2
                rb = sb * ROWS

                def row_body(r, _):
                    cf = vbuf[pl.ds(rb + r, 16)][0]
                    dl = dlbuf[pl.ds(rb + r, 16)][0]
                    for k in range(D // 16):
                        sl = pl.ds(k * 16, 16)
                        plsc.addupdate(acc.at[dl, sl], rowbuf[par, r, sl] * cf)
                    return 0
                lax.fori_loop(0, jnp.minimum(ROWS, ne - rb), row_body, 0)
                return 0
            lax.fori_loop(0, nsub, sub_body, 0)
            return 0
        lax.fori_loop(0, nch, chunk_body, 0)

        def red_sum(c, _):
            v = priv[pl.ds(c * 16, 16)]
            for l in range(1, 16):
                v = v + priv[pl.ds(l * NL + c * 16, 16)]
            s[pl.ds(c * 16, 16)] = 1.0 / (v + 1e-16)
            return 0
        lax.fori_loop(0, NL // 16, red_sum, 0)

        def out_body(i, _):
            si = s[pl.ds(i, 16)][0]
            for k in range(D // 16):
                sl = pl.ds(k * 16, 16)
                acc[i, sl] = jnp.maximum(acc[i, sl] * si + biasl[sl], 0.0)
            return 0
        lax.fori_loop(0, NL, out_body, 0)

        pltpu.sync_copy(acc, out_hbm.at[pl.ds(base, NL)])

    return aggregate


_aggregate_256 = _make_aggregate(256)
_aggregate_128 = _make_aggregate(128)


def kernel(x, edge_index, W1, a_src1, a_dst1, b1, W2, a_src2, a_dst2, b2):
    ei = edge_index.astype(jnp.int32)
    bedges, counts = _bucketize(ei)

    W1e = jnp.concatenate(
        [W1, (W1 @ a_src1)[:, None], (W1 @ a_dst1)[:, None]], axis=1)
    ze1 = _matmul(x, W1e)
    z1 = ze1[:, :256]
    as1 = ze1[:, 256]
    ad1 = jnp.pad(ze1[:, 257], (0, NPAD - N))
    h1 = _aggregate_256(z1, as1, ad1, bedges, counts, b1)[:N]

    W2e = jnp.concatenate(
        [W2, (W2 @ a_src2)[:, None], (W2 @ a_dst2)[:, None]], axis=1)
    ze2 = _matmul(h1, W2e)
    z2 = ze2[:, :128]
    as2 = ze2[:, 128]
    ad2 = jnp.pad(ze2[:, 129], (0, NPAD - N))
    h2 = _aggregate_128(z2, as2, ad2, bedges, counts, b2)[:N]
    return h2
